# Initial kernel scaffold; baseline (speedup 1.0000x reference)
import functools

import jax
import jax.numpy as jnp
from jax.experimental import pallas as pl
from jax.experimental.pallas import tpu as pltpu
from jax.experimental.pallas import tpu_sc as plsc

B, N = 4096, 200
TOTAL = B * N
EMB_DIM = 32
NUM_DIM = 16
OUT_DIM = 64

GATHER_WINDOW = 128
ROW_BLOCK = 8192


def _sc_gather(emb_table, idx_2d):
    mesh = plsc.VectorSubcoreMesh(core_axis_name="c", subcore_axis_name="s")

    @functools.partial(
        pl.kernel,
        out_type=jax.ShapeDtypeStruct((TOTAL, EMB_DIM), jnp.float32),
        mesh=mesh,
    )
    def gather_kernel(table_hbm, idx_hbm, out_hbm):
        def body(i_vmem, o_vmem):
            pltpu.sync_copy(table_hbm.at[i_vmem.at[0]], o_vmem)

        pltpu.emit_pipeline(
            body,
            grid=(TOTAL // GATHER_WINDOW,),
            in_specs=[pl.BlockSpec((1, GATHER_WINDOW), lambda i: (0, i))],
            out_specs=[pl.BlockSpec((GATHER_WINDOW, EMB_DIM),
                                    lambda i: (i, 0))],
            core_axis_name=("c", "s"),
            dimension_semantics=(pltpu.PARALLEL,),
        )(idx_hbm, out_hbm)

    return gather_kernel(emb_table, idx_2d)


def _tc_project(gathered, numeric_flat, We, Wn, b2d):
    def body(g_ref, n_ref, we_ref, wn_ref, b_ref, o_ref):
        acc = jnp.dot(g_ref[...], we_ref[...],
                      preferred_element_type=jnp.float32)
        acc += jnp.dot(n_ref[...], wn_ref[...],
                       preferred_element_type=jnp.float32)
        o_ref[...] = jnp.maximum(acc + b_ref[...], 0.0)

    return pl.pallas_call(
        body,
        grid=(TOTAL // ROW_BLOCK,),
        in_specs=[
            pl.BlockSpec((ROW_BLOCK, EMB_DIM), lambda i: (i, 0)),
            pl.BlockSpec((ROW_BLOCK, NUM_DIM), lambda i: (i, 0)),
            pl.BlockSpec((EMB_DIM, OUT_DIM), lambda i: (0, 0)),
            pl.BlockSpec((NUM_DIM, OUT_DIM), lambda i: (0, 0)),
            pl.BlockSpec((1, OUT_DIM), lambda i: (0, 0)),
        ],
        out_specs=pl.BlockSpec((ROW_BLOCK, OUT_DIM), lambda i: (i, 0)),
        out_shape=jax.ShapeDtypeStruct((TOTAL, OUT_DIM), jnp.float32),
    )(gathered, numeric_flat, We, Wn, b2d)


def kernel(unit_types, numeric, emb_table, W, b):
    idx_2d = unit_types.astype(jnp.int32).reshape(1, TOTAL)
    gathered = _sc_gather(emb_table, idx_2d)
    We = W[:, :EMB_DIM].T
    Wn = W[:, EMB_DIM:].T
    numeric_flat = numeric.reshape(TOTAL, NUM_DIM)
    out = _tc_project(gathered, numeric_flat, We, Wn, b.reshape(1, OUT_DIM))
    return out.reshape(B, N, OUT_DIM)


# speedup vs baseline: 2.6960x; 2.6960x over previous
import functools

import jax
import jax.numpy as jnp
from jax.experimental import pallas as pl
from jax.experimental.pallas import tpu as pltpu
from jax.experimental.pallas import tpu_sc as plsc

B, N = 4096, 200
TOTAL = B * N
EMB_DIM = 32
NUM_DIM = 16
OUT_DIM = 64

GATHER_WINDOW = 128
ROW_BLOCK = 8192


def _sc_gather(emb_table, idx_2d):
    mesh = plsc.VectorSubcoreMesh(core_axis_name="c", subcore_axis_name="s")

    @functools.partial(
        pl.kernel,
        out_type=jax.ShapeDtypeStruct((TOTAL, EMB_DIM), jnp.float32),
        mesh=mesh,
        compiler_params=pltpu.CompilerParams(use_tc_tiling_on_sc=False),
    )
    def gather_kernel(table_hbm, idx_hbm, out_hbm):
        def body(i_vmem, o_vmem):
            pltpu.sync_copy(table_hbm.at[i_vmem.at[0]], o_vmem)

        pltpu.emit_pipeline(
            body,
            grid=(TOTAL // GATHER_WINDOW,),
            in_specs=[pl.BlockSpec((1, GATHER_WINDOW), lambda i: (0, i))],
            out_specs=[pl.BlockSpec((GATHER_WINDOW, EMB_DIM),
                                    lambda i: (i, 0))],
            core_axis_name=("c", "s"),
            dimension_semantics=(pltpu.PARALLEL,),
        )(idx_hbm, out_hbm)

    return gather_kernel(emb_table, idx_2d)


def _tc_project(gathered, numeric_flat, We, Wn, b2d):
    def body(g_ref, n_ref, we_ref, wn_ref, b_ref, o_ref):
        acc = jnp.dot(g_ref[...], we_ref[...],
                      preferred_element_type=jnp.float32)
        acc += jnp.dot(n_ref[...], wn_ref[...],
                       preferred_element_type=jnp.float32)
        o_ref[...] = jnp.maximum(acc + b_ref[...], 0.0)

    return pl.pallas_call(
        body,
        grid=(TOTAL // ROW_BLOCK,),
        in_specs=[
            pl.BlockSpec((ROW_BLOCK, EMB_DIM), lambda i: (i, 0)),
            pl.BlockSpec((ROW_BLOCK, NUM_DIM), lambda i: (i, 0)),
            pl.BlockSpec((EMB_DIM, OUT_DIM), lambda i: (0, 0)),
            pl.BlockSpec((NUM_DIM, OUT_DIM), lambda i: (0, 0)),
            pl.BlockSpec((1, OUT_DIM), lambda i: (0, 0)),
        ],
        out_specs=pl.BlockSpec((ROW_BLOCK, OUT_DIM), lambda i: (i, 0)),
        out_shape=jax.ShapeDtypeStruct((TOTAL, OUT_DIM), jnp.float32),
    )(gathered, numeric_flat, We, Wn, b2d)


def kernel(unit_types, numeric, emb_table, W, b):
    idx_2d = unit_types.astype(jnp.int32).reshape(1, TOTAL)
    gathered = _sc_gather(emb_table, idx_2d)
    We = W[:, :EMB_DIM].T
    Wn = W[:, EMB_DIM:].T
    numeric_flat = numeric.reshape(TOTAL, NUM_DIM)
    out = _tc_project(gathered, numeric_flat, We, Wn, b.reshape(1, OUT_DIM))
    return out.reshape(B, N, OUT_DIM)


# speedup vs baseline: 2.9800x; 1.1053x over previous
import functools

import jax
import jax.numpy as jnp
from jax.experimental import pallas as pl
from jax.experimental.pallas import tpu as pltpu
from jax.experimental.pallas import tpu_sc as plsc

B, N = 4096, 200
TOTAL = B * N
EMB_DIM = 32
NUM_DIM = 16
OUT_DIM = 64

GATHER_WINDOW = 1024
ROW_BLOCK = 8192


def _sc_gather(emb_table, idx_2d):
    mesh = plsc.VectorSubcoreMesh(core_axis_name="c", subcore_axis_name="s")

    @functools.partial(
        pl.kernel,
        out_type=jax.ShapeDtypeStruct((TOTAL, EMB_DIM), jnp.float32),
        mesh=mesh,
        compiler_params=pltpu.CompilerParams(use_tc_tiling_on_sc=False),
    )
    def gather_kernel(table_hbm, idx_hbm, out_hbm):
        def body(i_vmem, o_vmem):
            pltpu.sync_copy(table_hbm.at[i_vmem.at[0]], o_vmem)

        pltpu.emit_pipeline(
            body,
            grid=(TOTAL // GATHER_WINDOW,),
            in_specs=[pl.BlockSpec((1, GATHER_WINDOW), lambda i: (0, i))],
            out_specs=[pl.BlockSpec((GATHER_WINDOW, EMB_DIM),
                                    lambda i: (i, 0))],
            core_axis_name=("c", "s"),
            dimension_semantics=(pltpu.PARALLEL,),
        )(idx_hbm, out_hbm)

    return gather_kernel(emb_table, idx_2d)


def _tc_project(gathered, numeric_flat, We, Wn, b2d):
    def body(g_ref, n_ref, we_ref, wn_ref, b_ref, o_ref):
        acc = jnp.dot(g_ref[...], we_ref[...],
                      preferred_element_type=jnp.float32)
        acc += jnp.dot(n_ref[...], wn_ref[...],
                       preferred_element_type=jnp.float32)
        o_ref[...] = jnp.maximum(acc + b_ref[...], 0.0)

    return pl.pallas_call(
        body,
        grid=(TOTAL // ROW_BLOCK,),
        in_specs=[
            pl.BlockSpec((ROW_BLOCK, EMB_DIM), lambda i: (i, 0)),
            pl.BlockSpec((ROW_BLOCK, NUM_DIM), lambda i: (i, 0)),
            pl.BlockSpec((EMB_DIM, OUT_DIM), lambda i: (0, 0)),
            pl.BlockSpec((NUM_DIM, OUT_DIM), lambda i: (0, 0)),
            pl.BlockSpec((1, OUT_DIM), lambda i: (0, 0)),
        ],
        out_specs=pl.BlockSpec((ROW_BLOCK, OUT_DIM), lambda i: (i, 0)),
        out_shape=jax.ShapeDtypeStruct((TOTAL, OUT_DIM), jnp.float32),
    )(gathered, numeric_flat, We, Wn, b2d)


def kernel(unit_types, numeric, emb_table, W, b):
    idx_2d = unit_types.astype(jnp.int32).reshape(1, TOTAL)
    gathered = _sc_gather(emb_table, idx_2d)
    We = W[:, :EMB_DIM].T
    Wn = W[:, EMB_DIM:].T
    numeric_flat = numeric.reshape(TOTAL, NUM_DIM)
    out = _tc_project(gathered, numeric_flat, We, Wn, b.reshape(1, OUT_DIM))
    return out.reshape(B, N, OUT_DIM)


# speedup vs baseline: 4.5491x; 1.5265x over previous
import functools

import jax
import jax.numpy as jnp
from jax.experimental import pallas as pl
from jax.experimental.pallas import tpu as pltpu
from jax.experimental.pallas import tpu_sc as plsc

B, N = 4096, 200
TOTAL = B * N
EMB_DIM = 32
NUM_DIM = 16
OUT_DIM = 64

PACK = 128 // EMB_DIM
K_ROWS = 3200
NB = TOTAL // (PACK * K_ROWS)
SUB = 4
C_ROWS = K_ROWS // SUB


def _sc_gather_packed(emb_table, idx_2d):
    mesh = plsc.VectorSubcoreMesh(core_axis_name="c", subcore_axis_name="s")

    @functools.partial(
        pl.kernel,
        out_type=jax.ShapeDtypeStruct((TOTAL // PACK, 128), jnp.float32),
        mesh=mesh,
        compiler_params=pltpu.CompilerParams(use_tc_tiling_on_sc=False),
    )
    def gather_kernel(table_hbm, idx_hbm, out_hbm):
        def body(i_vmem, o_vmem):
            pltpu.sync_copy(table_hbm.at[i_vmem.at[0]], o_vmem)

        pltpu.emit_pipeline(
            body,
            grid=(NB * PACK * SUB,),
            in_specs=[pl.BlockSpec((1, C_ROWS), lambda t: (0, t))],
            out_specs=[pl.BlockSpec(
                (C_ROWS, EMB_DIM),
                lambda t: ((t // (PACK * SUB)) * SUB + t % SUB,
                           (t // SUB) % PACK))],
            core_axis_name=("c", "s"),
            dimension_semantics=(pltpu.PARALLEL,),
        )(idx_hbm, out_hbm)

    return gather_kernel(emb_table, idx_2d)


def _tc_project(packed, numeric_flat, We, Wn, b2d):
    rows = PACK * K_ROWS

    def body(g_ref, n_ref, we_ref, wn_ref, b_ref, o_ref):
        acc = jnp.dot(n_ref[...], wn_ref[...],
                      preferred_element_type=jnp.float32) + b_ref[...]
        for j in range(PACK):
            g = g_ref[:, pl.ds(j * EMB_DIM, EMB_DIM)]
            o = jnp.dot(g, we_ref[...], preferred_element_type=jnp.float32)
            a = jax.lax.slice(acc, (j * K_ROWS, 0), ((j + 1) * K_ROWS, OUT_DIM))
            o_ref[pl.ds(j * K_ROWS, K_ROWS), :] = jnp.maximum(o + a, 0.0)

    return pl.pallas_call(
        body,
        grid=(NB,),
        in_specs=[
            pl.BlockSpec((K_ROWS, 128), lambda i: (i, 0)),
            pl.BlockSpec((rows, NUM_DIM), lambda i: (i, 0)),
            pl.BlockSpec((EMB_DIM, OUT_DIM), lambda i: (0, 0)),
            pl.BlockSpec((NUM_DIM, OUT_DIM), lambda i: (0, 0)),
            pl.BlockSpec((1, OUT_DIM), lambda i: (0, 0)),
        ],
        out_specs=pl.BlockSpec((rows, OUT_DIM), lambda i: (i, 0)),
        out_shape=jax.ShapeDtypeStruct((TOTAL, OUT_DIM), jnp.float32),
    )(packed, numeric_flat, We, Wn, b2d)


def kernel(unit_types, numeric, emb_table, W, b):
    idx_2d = unit_types.astype(jnp.int32).reshape(1, TOTAL)
    packed = _sc_gather_packed(emb_table, idx_2d)
    We = W[:, :EMB_DIM].T
    Wn = W[:, EMB_DIM:].T
    numeric_flat = numeric.reshape(TOTAL, NUM_DIM)
    out = _tc_project(packed, numeric_flat, We, Wn, b.reshape(1, OUT_DIM))
    return out.reshape(B, N, OUT_DIM)


# speedup vs baseline: 10.7016x; 2.3525x over previous
import functools

import jax
import jax.numpy as jnp
from jax.experimental import pallas as pl
from jax.experimental.pallas import tpu as pltpu
from jax.experimental.pallas import tpu_sc as plsc

B, N = 4096, 200
TOTAL = B * N
EMB_DIM = 32
NUM_DIM = 16
OUT_DIM = 64

PACK = 128 // EMB_DIM
PR = B // PACK
P_BLK = 8


def _sc_gather_packed(emb_table, idx_2d):
    mesh = plsc.VectorSubcoreMesh(core_axis_name="c", subcore_axis_name="s")

    @functools.partial(
        pl.kernel,
        out_type=jax.ShapeDtypeStruct((N * PR, 128), jnp.float32),
        mesh=mesh,
        compiler_params=pltpu.CompilerParams(use_tc_tiling_on_sc=False),
    )
    def gather_kernel(table_hbm, idx_hbm, out_hbm):
        def body(i_vmem, o_vmem):
            pltpu.sync_copy(table_hbm.at[i_vmem.at[0]], o_vmem)

        pltpu.emit_pipeline(
            body,
            grid=(N * PACK,),
            in_specs=[pl.BlockSpec((1, PR), lambda t: (0, t))],
            out_specs=[pl.BlockSpec((PR, EMB_DIM),
                                    lambda t: (t // PACK, t % PACK))],
            core_axis_name=("c", "s"),
            dimension_semantics=(pltpu.PARALLEL,),
        )(idx_hbm, out_hbm)

    return gather_kernel(emb_table, idx_2d)


def _tc_project_t(packed, n_t, Wm, b_col):
    def body(e_ref, n_ref, w_ref, b_ref, o_ref):
        for q in range(P_BLK):
            accn = jax.lax.dot_general(
                w_ref[:, EMB_DIM:], n_ref[q],
                (((1,), (0,)), ((), ())),
                preferred_element_type=jnp.float32)
            for j in range(PACK):
                g = e_ref[pl.ds(q * PR, PR), pl.ds(j * EMB_DIM, EMB_DIM)]
                og = jax.lax.dot_general(
                    w_ref[:, :EMB_DIM], g,
                    (((1,), (1,)), ((), ())),
                    preferred_element_type=jnp.float32)
                a = jax.lax.slice(accn, (0, j * PR), (OUT_DIM, (j + 1) * PR))
                o_ref[q, :, pl.ds(j * PR, PR)] = jnp.maximum(
                    og + a + b_ref[...], 0.0)

    return pl.pallas_call(
        body,
        grid=(N // P_BLK,),
        in_specs=[
            pl.BlockSpec((P_BLK * PR, 128), lambda i: (i, 0)),
            pl.BlockSpec((P_BLK, NUM_DIM, B), lambda i: (i, 0, 0)),
            pl.BlockSpec((OUT_DIM, EMB_DIM + NUM_DIM), lambda i: (0, 0)),
            pl.BlockSpec((OUT_DIM, 1), lambda i: (0, 0)),
        ],
        out_specs=pl.BlockSpec((P_BLK, OUT_DIM, B), lambda i: (i, 0, 0)),
        out_shape=jax.ShapeDtypeStruct((N, OUT_DIM, B), jnp.float32),
    )(packed, n_t, Wm, b_col)


def kernel(unit_types, numeric, emb_table, W, b):
    idx_2d = unit_types.astype(jnp.int32).T.reshape(1, TOTAL)
    packed = _sc_gather_packed(emb_table, idx_2d)
    n_t = jnp.transpose(numeric, (1, 2, 0))
    out_t = _tc_project_t(packed, n_t, W, b.reshape(OUT_DIM, 1))
    return jnp.transpose(out_t, (2, 0, 1))
